# baseline (speedup 1.0000x reference)
import functools

import jax
import jax.numpy as jnp
from jax import lax
from jax.experimental import pallas as pl
from jax.experimental.pallas import tpu as pltpu
from jax.experimental.pallas import tpu_sc as plsc

N, H, W, C = 2048, 32, 32, 16
NUM_WORKERS = 32
IMGS_PER_WORKER = N // NUM_WORKERS


def _body(x_hbm, crd_hbm, out_hbm, img_v, out_v, crd_v, tabi_s, tabf_s):
    wid = lax.axis_index("s") * 2 + lax.axis_index("c")
    base = wid * IMGS_PER_WORKER

    def one_image(k, carry):
        n = base + k
        pltpu.sync_copy(crd_hbm.at[n], crd_v)
        pltpu.sync_copy(x_hbm.at[n], img_v)

        for axis in range(2):
            for ck in range(2):
                v = crd_v[pl.ds(32 * axis + 16 * ck, 16)]
                Xc = (v + 1.0) * 16.0
                ti = Xc.astype(jnp.int32)
                x0 = jnp.where(ti.astype(jnp.float32) > Xc, ti - 1, ti)
                x0c = jnp.clip(x0, 0, 31)
                x1c = jnp.clip(x0 + 1, 0, 31)
                w0 = x1c.astype(jnp.float32) - Xc
                w1 = Xc - x0c.astype(jnp.float32)
                if axis == 0:
                    x0c = x0c * C
                    x1c = x1c * C
                else:
                    x0c = x0c * (W * C)
                    x1c = x1c * (W * C)
                r0, r1 = (0, 1) if axis == 0 else (2, 3)
                for lane in range(16):
                    col = 16 * ck + lane
                    tabi_s[r0, col] = x0c[lane]
                    tabi_s[r1, col] = x1c[lane]
                    tabf_s[r0, col] = w0[lane]
                    tabf_s[r1, col] = w1[lane]

        @plsc.parallel_loop(0, H)
        def row_body(i):
            y0r = tabi_s[2, i]
            y1r = tabi_s[3, i]
            wy0 = tabf_s[2, i]
            wy1 = tabf_s[3, i]

            rowbase = i * (W * C)

            @plsc.parallel_loop(0, W, unroll=16)
            def col_body(j):
                x0 = tabi_s[0, j]
                x1 = tabi_s[1, j]
                wx0 = tabf_s[0, j]
                wx1 = tabf_s[1, j]
                a = img_v[pl.ds(y0r + x0, C)]
                b = img_v[pl.ds(y1r + x0, C)]
                c = img_v[pl.ds(y0r + x1, C)]
                d = img_v[pl.ds(y1r + x1, C)]
                out_v[pl.ds(rowbase + j * C, C)] = wy0 * (
                    wx0 * a + wx1 * c
                ) + wy1 * (wx0 * b + wx1 * d)


        pltpu.sync_copy(out_v, out_hbm.at[n])
        return carry

    lax.fori_loop(0, IMGS_PER_WORKER, one_image, 0)


@jax.jit
def kernel(X, theta):
    x2 = X.reshape(N, H * W * C)

    t0, t1, t2 = theta[:, 0], theta[:, 1], theta[:, 2]
    zr = jnp.zeros_like(t0)
    Ts = jnp.stack(
        [jnp.stack([t0, zr, t1], -1), jnp.stack([zr, t0, t2], -1)], -2
    )
    Xt, Yt = jnp.meshgrid(jnp.linspace(-1, 1, W), jnp.linspace(-1, 1, H))
    Gt = jnp.vstack([Xt.flatten(), Yt.flatten(), jnp.ones(Xt.size)])
    Gs = jax.vmap(lambda A: A @ Gt)(Ts)
    xs = Gs[:, 0, :W]
    ys = Gs[:, 1, ::W]
    coords = jnp.concatenate([xs, ys], axis=1)

    mesh = plsc.VectorSubcoreMesh(core_axis_name="c", subcore_axis_name="s")
    run = pl.kernel(
        _body,
        out_type=jax.ShapeDtypeStruct((N, H * W * C), jnp.float32),
        mesh=mesh,
        scratch_types=[
            pltpu.VMEM((H * W * C,), jnp.float32),
            pltpu.VMEM((H * W * C,), jnp.float32),
            pltpu.VMEM((64,), jnp.float32),
            pltpu.SMEM((4, 32), jnp.int32),
            pltpu.SMEM((4, 32), jnp.float32),
        ],
    )
    out = run(x2, coords)
    return out.reshape(N, H, W, C)


# speedup vs baseline: 1.4416x; 1.4416x over previous
import functools

import jax
import jax.numpy as jnp
from jax import lax
from jax.experimental import pallas as pl
from jax.experimental.pallas import tpu as pltpu
from jax.experimental.pallas import tpu_sc as plsc

N, H, W, C = 2048, 32, 32, 16
NUM_WORKERS = 32
IMGS_PER_WORKER = N // NUM_WORKERS


def _body(x_hbm, crd_hbm, out_hbm, img_v, out_v, crd_v, tabi_s, tabf_s):
    wid = lax.axis_index("s") * 2 + lax.axis_index("c")
    base = wid * IMGS_PER_WORKER

    def one_image(k, carry):
        n = base + k
        pltpu.sync_copy(crd_hbm.at[n], crd_v)
        pltpu.sync_copy(x_hbm.at[n], img_v)

        for axis in range(2):
            for ck in range(2):
                v = crd_v[pl.ds(32 * axis + 16 * ck, 16)]
                Xc = (v + 1.0) * 16.0
                ti = Xc.astype(jnp.int32)
                x0 = jnp.where(ti.astype(jnp.float32) > Xc, ti - 1, ti)
                x0c = jnp.clip(x0, 0, 31)
                x1c = jnp.clip(x0 + 1, 0, 31)
                w0 = x1c.astype(jnp.float32) - Xc
                w1 = Xc - x0c.astype(jnp.float32)
                if axis == 0:
                    x0c = x0c * C
                    x1c = x1c * C
                else:
                    x0c = x0c * (W * C)
                    x1c = x1c * (W * C)
                r0, r1 = (0, 1) if axis == 0 else (2, 3)
                for lane in range(16):
                    col = 16 * ck + lane
                    tabi_s[r0, col] = x0c[lane]
                    tabi_s[r1, col] = x1c[lane]
                    tabf_s[r0, col] = w0[lane]
                    tabf_s[r1, col] = w1[lane]

        def row_body(i, carry):
            y0r = tabi_s[2, i]
            y1r = tabi_s[3, i]
            wy0 = tabf_s[2, i]
            wy1 = tabf_s[3, i]

            rowbase = i * (W * C)

            @plsc.parallel_loop(0, W, unroll=16)
            def col_body(j):
                x0 = tabi_s[0, j]
                x1 = tabi_s[1, j]
                wx0 = tabf_s[0, j]
                wx1 = tabf_s[1, j]
                a = img_v[pl.ds(y0r + x0, C)]
                b = img_v[pl.ds(y1r + x0, C)]
                c = img_v[pl.ds(y0r + x1, C)]
                d = img_v[pl.ds(y1r + x1, C)]
                out_v[pl.ds(rowbase + j * C, C)] = wy0 * (
                    wx0 * a + wx1 * c
                ) + wy1 * (wx0 * b + wx1 * d)


        pltpu.sync_copy(out_v, out_hbm.at[n])
        return carry

    lax.fori_loop(0, IMGS_PER_WORKER, one_image, 0)


@jax.jit
def kernel(X, theta):
    x2 = X.reshape(N, H * W * C)

    t0, t1, t2 = theta[:, 0], theta[:, 1], theta[:, 2]
    zr = jnp.zeros_like(t0)
    Ts = jnp.stack(
        [jnp.stack([t0, zr, t1], -1), jnp.stack([zr, t0, t2], -1)], -2
    )
    Xt, Yt = jnp.meshgrid(jnp.linspace(-1, 1, W), jnp.linspace(-1, 1, H))
    Gt = jnp.vstack([Xt.flatten(), Yt.flatten(), jnp.ones(Xt.size)])
    Gs = jax.vmap(lambda A: A @ Gt)(Ts)
    xs = Gs[:, 0, :W]
    ys = Gs[:, 1, ::W]
    coords = jnp.concatenate([xs, ys], axis=1)

    mesh = plsc.VectorSubcoreMesh(core_axis_name="c", subcore_axis_name="s")
    run = pl.kernel(
        _body,
        out_type=jax.ShapeDtypeStruct((N, H * W * C), jnp.float32),
        mesh=mesh,
        scratch_types=[
            pltpu.VMEM((H * W * C,), jnp.float32),
            pltpu.VMEM((H * W * C,), jnp.float32),
            pltpu.VMEM((64,), jnp.float32),
            pltpu.SMEM((4, 32), jnp.int32),
            pltpu.SMEM((4, 32), jnp.float32),
        ],
    )
    out = run(x2, coords)
    return out.reshape(N, H, W, C)
